# baseline (speedup 1.0000x reference)
import functools

import jax
import jax.numpy as jnp
from jax import lax
from jax.experimental import pallas as pl
from jax.experimental.pallas import tpu as pltpu
from jax.experimental.pallas import tpu_sc as plsc

N_USERS = 25000
N_ITEMS = 25000
N = N_USERS + N_ITEMS
D = 64
E = 800000
LAYERS = 4

NC, NS, L = 2, 16, 16
HALF = 25088
NPAD = 2 * HALF
ROWS_T = HALF // NS
GARB = 16
ACC_ROWS = HALF + GARB
ZROWS_T = ACC_ROWS // NS
EC = 128
ES = 50176
EPAD = ES * NS
NCH_E = ES // EC
BC = 8
NB = NCH_E // BC
NCH_ALL = EPAD // EC
KQ = 2
ZB = 16
RC = 28
BLK = 512


def _deg_body(dst_hbm, deg16_hbm, idxl_hbm, acc16, ones_v, zb16, dstb, idxb):
    c = lax.axis_index("c")
    s = lax.axis_index("s")

    @pl.loop(0, EC)
    def _fill(i):
        ones_v[i, :] = jnp.ones((L,), jnp.float32)

    @pl.loop(0, ZB)
    def _fillz(i):
        zb16[i, :] = jnp.zeros((L,), jnp.float32)

    z0r = s * ZROWS_T

    @pl.loop(0, ZROWS_T // ZB)
    def _zero(i):
        pltpu.sync_copy(zb16, acc16.at[pl.ds(z0r + i * ZB, ZB)])

    _ztail = ZROWS_T - (ZROWS_T // ZB) * ZB
    pltpu.sync_copy(zb16.at[pl.ds(0, _ztail)],
                    acc16.at[pl.ds(z0r + (ZROWS_T // ZB) * ZB, _ztail)])
    plsc.subcore_barrier()

    lo = c * HALF
    garb = HALF + s

    @pl.loop(0, NB)
    def _sweep(t):
        cbase = s * NCH_E + t * BC
        pltpu.sync_copy(dst_hbm.at[pl.ds(cbase, BC)], dstb)
        for k in range(BC):
            for j in range(EC // L):
                d = dstb[k, pl.ds(j * L, L)]
                inh = (d >= lo) & (d < lo + HALF)
                idxb[k, pl.ds(j * L, L)] = jnp.where(inh, d - lo, garb)
            pltpu.sync_copy(ones_v, acc16.at[idxb.at[k]], add=True)
        pltpu.sync_copy(idxb, idxl_hbm.at[c, pl.ds(cbase, BC)])

    plsc.subcore_barrier()

    rbase = s * ROWS_T
    gbase = c * HALF + s * ROWS_T
    pltpu.sync_copy(acc16.at[pl.ds(rbase, ROWS_T)],
                    deg16_hbm.at[pl.ds(gbase, ROWS_T)])


def _layer_body(src_hbm, idxl_hbm, z_hbm, dinvrow_hbm, tot_hbm,
                z_out, tot_out,
                acc, zb, srcb, idxb, rows, abuf, dbuf, tbuf,
                sem_g, sem_s0, sem_s1, sem_i):
    c = lax.axis_index("c")
    s = lax.axis_index("s")

    @pl.loop(0, ZB)
    def _fill(i):
        for j in range(D // L):
            zb[i, pl.ds(j * L, L)] = jnp.zeros((L,), jnp.float32)

    z0r = s * ZROWS_T

    @pl.loop(0, ZROWS_T // ZB)
    def _zero(i):
        pltpu.sync_copy(zb, acc.at[pl.ds(z0r + i * ZB, ZB)])

    _ztail = ZROWS_T - (ZROWS_T // ZB) * ZB
    pltpu.sync_copy(zb.at[pl.ds(0, _ztail)],
                    acc.at[pl.ds(z0r + (ZROWS_T // ZB) * ZB, _ztail)])
    plsc.subcore_barrier()

    cb0 = s * NCH_E
    pltpu.async_copy(src_hbm.at[pl.ds(cb0, BC)], srcb.at[0], sem_i)
    pltpu.async_copy(idxl_hbm.at[c, pl.ds(cb0, BC)], idxb.at[0], sem_i)

    sem_s = (sem_s0, sem_s1)

    @pl.loop(0, NB)
    def _sweep(t):
        ib = t % 2
        nib = (t + 1) % 2
        cbase = s * NCH_E + t * BC
        pltpu.make_async_copy(
            src_hbm.at[pl.ds(cbase, BC)], srcb.at[ib], sem_i).wait()
        pltpu.make_async_copy(
            idxl_hbm.at[c, pl.ds(cbase, BC)], idxb.at[ib], sem_i).wait()

        @pl.when(t + 1 < NB)
        def _prefetch():
            nbase = s * NCH_E + (t + 1) * BC
            pltpu.async_copy(src_hbm.at[pl.ds(nbase, BC)], srcb.at[nib], sem_i)
            pltpu.async_copy(
                idxl_hbm.at[c, pl.ds(nbase, BC)], idxb.at[nib], sem_i)

        for k in range(BC):
            b = k % KQ
            if k >= KQ:
                pltpu.make_async_copy(
                    rows.at[b], acc.at[idxb.at[ib, k - KQ]], sem_s[b]).wait()
            pltpu.async_copy(z_hbm.at[srcb.at[ib, k]], rows.at[b], sem_g).wait()
            pltpu.async_copy(rows.at[b], acc.at[idxb.at[ib, k]],
                             sem_s[b], add=True)
        for k in range(BC - KQ, BC):
            b = k % KQ
            pltpu.make_async_copy(
                rows.at[b], acc.at[idxb.at[ib, k]], sem_s[b]).wait()

    plsc.subcore_barrier()

    rbase = s * ROWS_T
    gbase = c * HALF + s * ROWS_T

    @pl.loop(0, ROWS_T // RC)
    def _out(k):
        g0 = gbase + k * RC
        pltpu.sync_copy(acc.at[pl.ds(rbase + k * RC, RC)], abuf)
        pltpu.sync_copy(dinvrow_hbm.at[pl.ds(g0, RC)], dbuf)
        pltpu.sync_copy(tot_hbm.at[pl.ds(g0, RC)], tbuf)

        @pl.loop(0, RC)
        def _scale(r):
            for j in range(D // L):
                dr = dbuf[r, pl.ds(j * L, L)]
                emb = abuf[r, pl.ds(j * L, L)] * dr
                tbuf[r, pl.ds(j * L, L)] = tbuf[r, pl.ds(j * L, L)] + emb
                abuf[r, pl.ds(j * L, L)] = emb * dr

        pltpu.sync_copy(tbuf, tot_out.at[pl.ds(g0, RC)])
        pltpu.sync_copy(abuf, z_out.at[pl.ds(g0, RC)])


def _dinv_tc_body(deg16_ref, emb0_ref, z0_ref, dinvrow_ref):
    deg = deg16_ref[:, 0:1]
    dinv = jnp.where(deg > 0.0, lax.rsqrt(jnp.maximum(deg, 1e-12)), 0.0)
    dinvrow_ref[...] = jnp.broadcast_to(dinv, (BLK, D))
    z0_ref[...] = emb0_ref[...] * dinv


@functools.lru_cache(maxsize=None)
def _build_kernels():
    mesh = plsc.VectorSubcoreMesh(
        core_axis_name="c", subcore_axis_name="s",
        num_cores=NC, num_subcores=NS)
    deg_k = pl.kernel(
        _deg_body,
        out_type=(
            jax.ShapeDtypeStruct((NPAD, L), jnp.float32),
            jax.ShapeDtypeStruct((NC, NCH_ALL, EC), jnp.int32),
        ),
        mesh=mesh,
        compiler_params=pltpu.CompilerParams(use_tc_tiling_on_sc=False),
        scratch_types=(
            pltpu.MemorySpace.VMEM_SHARED((ACC_ROWS, L), jnp.float32),
            pltpu.VMEM((EC, L), jnp.float32),
            pltpu.VMEM((ZB, L), jnp.float32),
            pltpu.VMEM((BC, EC), jnp.int32),
            pltpu.VMEM((BC, EC), jnp.int32),
        ),
    )
    layer_k = pl.kernel(
        _layer_body,
        out_type=(
            jax.ShapeDtypeStruct((NPAD, D), jnp.float32),
            jax.ShapeDtypeStruct((NPAD, D), jnp.float32),
        ),
        mesh=mesh,
        compiler_params=pltpu.CompilerParams(use_tc_tiling_on_sc=False),
        scratch_types=(
            pltpu.MemorySpace.VMEM_SHARED((ACC_ROWS, D), jnp.float32),
            pltpu.VMEM((ZB, D), jnp.float32),
            pltpu.VMEM((2, BC, EC), jnp.int32),
            pltpu.VMEM((2, BC, EC), jnp.int32),
            pltpu.VMEM((KQ, EC, D), jnp.float32),
            pltpu.VMEM((RC, D), jnp.float32),
            pltpu.VMEM((RC, D), jnp.float32),
            pltpu.VMEM((RC, D), jnp.float32),
            pltpu.SemaphoreType.DMA,
            pltpu.SemaphoreType.DMA,
            pltpu.SemaphoreType.DMA,
            pltpu.SemaphoreType.DMA,
        ),
    )
    grid = (NPAD // BLK,)
    blk2 = pl.BlockSpec((BLK, D), lambda i: (i, 0))
    blk16 = pl.BlockSpec((BLK, L), lambda i: (i, 0))
    dinv_k = pl.pallas_call(
        _dinv_tc_body,
        grid=grid,
        in_specs=[blk16, blk2],
        out_specs=[blk2, blk2],
        out_shape=(
            jax.ShapeDtypeStruct((NPAD, D), jnp.float32),
            jax.ShapeDtypeStruct((NPAD, D), jnp.float32),
        ),
    )
    return deg_k, layer_k, dinv_k


def kernel(edge_index, user_weight, item_weight):
    src = edge_index[0].astype(jnp.int32)
    dst = edge_index[1].astype(jnp.int32)
    pe = EPAD - E
    pad_src = (jnp.arange(pe, dtype=jnp.int32) * 997) % N
    pad_dst = jnp.full((pe,), NPAD, dtype=jnp.int32)
    srcp = jnp.concatenate([src, pad_src]).reshape(NCH_ALL, EC)
    dstp = jnp.concatenate([dst, pad_dst]).reshape(NCH_ALL, EC)

    emb0 = jnp.concatenate([user_weight, item_weight], axis=0)
    emb0p = jnp.pad(emb0, ((0, NPAD - N), (0, 0)))

    deg_k, layer_k, dinv_k = _build_kernels()
    deg16, idxl = deg_k(dstp)
    z, dinvrow = dinv_k(deg16, emb0p)
    tot = emb0p
    for _ in range(LAYERS):
        z, tot = layer_k(srcp, idxl, z, dinvrow, tot)

    out = tot * (1.0 / ((LAYERS + 1) * (LAYERS + 1)))
    return out[:N_USERS], out[N_USERS:N]


# speedup vs baseline: 1.2036x; 1.2036x over previous
import functools

import jax
import jax.numpy as jnp
from jax import lax
from jax.experimental import pallas as pl
from jax.experimental.pallas import tpu as pltpu
from jax.experimental.pallas import tpu_sc as plsc

N_USERS = 25000
N_ITEMS = 25000
N = N_USERS + N_ITEMS
D = 64
E = 800000
LAYERS = 4

NC, NS, L = 2, 16, 16
HALF = 25088
NPAD = 2 * HALF
ROWS_T = HALF // NS
GARB = 16
ACC_ROWS = HALF + GARB
ZROWS_T = ACC_ROWS // NS
EC = 128
ES = 50176
EPAD = ES * NS
NCH_E = ES // EC
BC = 8
NB = NCH_E // BC
NCH_ALL = EPAD // EC
KQ = 2
ZB = 16
RC = 28
BLK = 512


def _deg_body(dst_hbm, deg16_hbm, idxl_hbm, acc16, ones_v, zb16, dstb, idxb):
    c = lax.axis_index("c")
    s = lax.axis_index("s")

    @pl.loop(0, EC)
    def _fill(i):
        ones_v[i, :] = jnp.ones((L,), jnp.float32)

    @pl.loop(0, ZB)
    def _fillz(i):
        zb16[i, :] = jnp.zeros((L,), jnp.float32)

    z0r = s * ZROWS_T

    @pl.loop(0, ZROWS_T // ZB)
    def _zero(i):
        pltpu.sync_copy(zb16, acc16.at[pl.ds(z0r + i * ZB, ZB)])

    _ztail = ZROWS_T - (ZROWS_T // ZB) * ZB
    pltpu.sync_copy(zb16.at[pl.ds(0, _ztail)],
                    acc16.at[pl.ds(z0r + (ZROWS_T // ZB) * ZB, _ztail)])
    plsc.subcore_barrier()

    lo = c * HALF
    garb = HALF + s

    @pl.loop(0, NB)
    def _sweep(t):
        cbase = s * NCH_E + t * BC
        pltpu.sync_copy(dst_hbm.at[pl.ds(cbase, BC)], dstb)
        for k in range(BC):
            for j in range(EC // L):
                d = dstb[k, pl.ds(j * L, L)]
                inh = (d >= lo) & (d < lo + HALF)
                idxb[k, pl.ds(j * L, L)] = jnp.where(inh, d - lo, garb)
            pltpu.sync_copy(ones_v, acc16.at[idxb.at[k]], add=True)
        pltpu.sync_copy(idxb, idxl_hbm.at[c, pl.ds(cbase, BC)])

    plsc.subcore_barrier()

    rbase = s * ROWS_T
    gbase = c * HALF + s * ROWS_T
    pltpu.sync_copy(acc16.at[pl.ds(rbase, ROWS_T)],
                    deg16_hbm.at[pl.ds(gbase, ROWS_T)])


def _layer_body(src_hbm, idxl_hbm, z_hbm, dinvrow_hbm, tot_hbm,
                z_out, tot_out,
                acc, zb, srcb, idxb, rows, abuf, dbuf, tbuf,
                sem_g, sem_s0, sem_s1, sem_i):
    c = lax.axis_index("c")
    s = lax.axis_index("s")

    @pl.loop(0, ZB)
    def _fill(i):
        for j in range(D // L):
            zb[i, pl.ds(j * L, L)] = jnp.zeros((L,), jnp.float32)

    z0r = s * ZROWS_T

    @pl.loop(0, ZROWS_T // ZB)
    def _zero(i):
        pltpu.sync_copy(zb, acc.at[pl.ds(z0r + i * ZB, ZB)])

    _ztail = ZROWS_T - (ZROWS_T // ZB) * ZB
    pltpu.sync_copy(zb.at[pl.ds(0, _ztail)],
                    acc.at[pl.ds(z0r + (ZROWS_T // ZB) * ZB, _ztail)])
    plsc.subcore_barrier()

    cb0 = s * NCH_E
    pltpu.async_copy(src_hbm.at[pl.ds(cb0, BC)], srcb.at[0], sem_i)
    pltpu.async_copy(idxl_hbm.at[c, pl.ds(cb0, BC)], idxb.at[0], sem_i)

    sem_s = (sem_s0, sem_s1)

    @pl.loop(0, NB)
    def _sweep(t):
        ib = t % 2
        nib = (t + 1) % 2
        cbase = s * NCH_E + t * BC
        pltpu.make_async_copy(
            src_hbm.at[pl.ds(cbase, BC)], srcb.at[ib], sem_i).wait()
        pltpu.make_async_copy(
            idxl_hbm.at[c, pl.ds(cbase, BC)], idxb.at[ib], sem_i).wait()

        @pl.when(t + 1 < NB)
        def _prefetch():
            nbase = s * NCH_E + (t + 1) * BC
            pltpu.async_copy(src_hbm.at[pl.ds(nbase, BC)], srcb.at[nib], sem_i)
            pltpu.async_copy(
                idxl_hbm.at[c, pl.ds(nbase, BC)], idxb.at[nib], sem_i)

        cps = [None] * BC
        for k in range(KQ):
            cps[k] = pltpu.async_copy(
                z_hbm.at[srcb.at[ib, k]], rows.at[k % KQ], sem_g)
        for k in range(BC):
            cps[k].wait()
            pltpu.sync_copy(rows.at[k % KQ], acc.at[idxb.at[ib, k]], add=True)
            nk = k + KQ
            if nk < BC:
                cps[nk] = pltpu.async_copy(
                    z_hbm.at[srcb.at[ib, nk]], rows.at[nk % KQ], sem_g)

    plsc.subcore_barrier()

    rbase = s * ROWS_T
    gbase = c * HALF + s * ROWS_T

    @pl.loop(0, ROWS_T // RC)
    def _out(k):
        g0 = gbase + k * RC
        pltpu.sync_copy(acc.at[pl.ds(rbase + k * RC, RC)], abuf)
        pltpu.sync_copy(dinvrow_hbm.at[pl.ds(g0, RC)], dbuf)
        pltpu.sync_copy(tot_hbm.at[pl.ds(g0, RC)], tbuf)

        @pl.loop(0, RC)
        def _scale(r):
            for j in range(D // L):
                dr = dbuf[r, pl.ds(j * L, L)]
                emb = abuf[r, pl.ds(j * L, L)] * dr
                tbuf[r, pl.ds(j * L, L)] = tbuf[r, pl.ds(j * L, L)] + emb
                abuf[r, pl.ds(j * L, L)] = emb * dr

        pltpu.sync_copy(tbuf, tot_out.at[pl.ds(g0, RC)])
        pltpu.sync_copy(abuf, z_out.at[pl.ds(g0, RC)])


def _dinv_tc_body(deg16_ref, emb0_ref, z0_ref, dinvrow_ref):
    deg = deg16_ref[:, 0:1]
    dinv = jnp.where(deg > 0.0, lax.rsqrt(jnp.maximum(deg, 1e-12)), 0.0)
    dinvrow_ref[...] = jnp.broadcast_to(dinv, (BLK, D))
    z0_ref[...] = emb0_ref[...] * dinv


@functools.lru_cache(maxsize=None)
def _build_kernels():
    mesh = plsc.VectorSubcoreMesh(
        core_axis_name="c", subcore_axis_name="s",
        num_cores=NC, num_subcores=NS)
    deg_k = pl.kernel(
        _deg_body,
        out_type=(
            jax.ShapeDtypeStruct((NPAD, L), jnp.float32),
            jax.ShapeDtypeStruct((NC, NCH_ALL, EC), jnp.int32),
        ),
        mesh=mesh,
        compiler_params=pltpu.CompilerParams(use_tc_tiling_on_sc=False),
        scratch_types=(
            pltpu.MemorySpace.VMEM_SHARED((ACC_ROWS, L), jnp.float32),
            pltpu.VMEM((EC, L), jnp.float32),
            pltpu.VMEM((ZB, L), jnp.float32),
            pltpu.VMEM((BC, EC), jnp.int32),
            pltpu.VMEM((BC, EC), jnp.int32),
        ),
    )
    layer_k = pl.kernel(
        _layer_body,
        out_type=(
            jax.ShapeDtypeStruct((NPAD, D), jnp.float32),
            jax.ShapeDtypeStruct((NPAD, D), jnp.float32),
        ),
        mesh=mesh,
        compiler_params=pltpu.CompilerParams(use_tc_tiling_on_sc=False),
        scratch_types=(
            pltpu.MemorySpace.VMEM_SHARED((ACC_ROWS, D), jnp.float32),
            pltpu.VMEM((ZB, D), jnp.float32),
            pltpu.VMEM((2, BC, EC), jnp.int32),
            pltpu.VMEM((2, BC, EC), jnp.int32),
            pltpu.VMEM((KQ, EC, D), jnp.float32),
            pltpu.VMEM((RC, D), jnp.float32),
            pltpu.VMEM((RC, D), jnp.float32),
            pltpu.VMEM((RC, D), jnp.float32),
            pltpu.SemaphoreType.DMA,
            pltpu.SemaphoreType.DMA,
            pltpu.SemaphoreType.DMA,
            pltpu.SemaphoreType.DMA,
        ),
    )
    grid = (NPAD // BLK,)
    blk2 = pl.BlockSpec((BLK, D), lambda i: (i, 0))
    blk16 = pl.BlockSpec((BLK, L), lambda i: (i, 0))
    dinv_k = pl.pallas_call(
        _dinv_tc_body,
        grid=grid,
        in_specs=[blk16, blk2],
        out_specs=[blk2, blk2],
        out_shape=(
            jax.ShapeDtypeStruct((NPAD, D), jnp.float32),
            jax.ShapeDtypeStruct((NPAD, D), jnp.float32),
        ),
    )
    return deg_k, layer_k, dinv_k


def kernel(edge_index, user_weight, item_weight):
    src = edge_index[0].astype(jnp.int32)
    dst = edge_index[1].astype(jnp.int32)
    pe = EPAD - E
    pad_src = (jnp.arange(pe, dtype=jnp.int32) * 997) % N
    pad_dst = jnp.full((pe,), NPAD, dtype=jnp.int32)
    srcp = jnp.concatenate([src, pad_src]).reshape(NCH_ALL, EC)
    dstp = jnp.concatenate([dst, pad_dst]).reshape(NCH_ALL, EC)

    emb0 = jnp.concatenate([user_weight, item_weight], axis=0)
    emb0p = jnp.pad(emb0, ((0, NPAD - N), (0, 0)))

    deg_k, layer_k, dinv_k = _build_kernels()
    deg16, idxl = deg_k(dstp)
    z, dinvrow = dinv_k(deg16, emb0p)
    tot = emb0p
    for _ in range(LAYERS):
        z, tot = layer_k(srcp, idxl, z, dinvrow, tot)

    out = tot * (1.0 / ((LAYERS + 1) * (LAYERS + 1)))
    return out[:N_USERS], out[N_USERS:N]


# speedup vs baseline: 1.5338x; 1.2744x over previous
import functools

import jax
import jax.numpy as jnp
from jax import lax
from jax.experimental import pallas as pl
from jax.experimental.pallas import tpu as pltpu
from jax.experimental.pallas import tpu_sc as plsc

N_USERS = 25000
N_ITEMS = 25000
N = N_USERS + N_ITEMS
D = 64
E = 800000
LAYERS = 4

NC, NS, L = 2, 16, 16
HALF = 25088
NPAD = 2 * HALF
ROWS_T = HALF // NS
GARB = 16
ACC_ROWS = HALF + GARB
ZROWS_T = ACC_ROWS // NS
EC = 128
ES = 50176
EPAD = ES * NS
NCH_E = ES // EC
BC = 8
NB = NCH_E // BC
NCH_ALL = EPAD // EC
KQ = 3
ZB = 16
RC = 28
BLK = 512


def _deg_body(dst_hbm, deg16_hbm, idxl_hbm, acc16, ones_v, zb16, dstb, idxb):
    c = lax.axis_index("c")
    s = lax.axis_index("s")

    @pl.loop(0, EC)
    def _fill(i):
        ones_v[i, :] = jnp.ones((L,), jnp.float32)

    @pl.loop(0, ZB)
    def _fillz(i):
        zb16[i, :] = jnp.zeros((L,), jnp.float32)

    z0r = s * ZROWS_T

    @pl.loop(0, ZROWS_T // ZB)
    def _zero(i):
        pltpu.sync_copy(zb16, acc16.at[pl.ds(z0r + i * ZB, ZB)])

    _ztail = ZROWS_T - (ZROWS_T // ZB) * ZB
    pltpu.sync_copy(zb16.at[pl.ds(0, _ztail)],
                    acc16.at[pl.ds(z0r + (ZROWS_T // ZB) * ZB, _ztail)])
    plsc.subcore_barrier()

    lo = c * HALF
    garb = HALF + s

    @pl.loop(0, NB)
    def _sweep(t):
        cbase = s * NCH_E + t * BC
        pltpu.sync_copy(dst_hbm.at[pl.ds(cbase, BC)], dstb)
        for k in range(BC):
            for j in range(EC // L):
                d = dstb[k, pl.ds(j * L, L)]
                inh = (d >= lo) & (d < lo + HALF)
                idxb[k, pl.ds(j * L, L)] = jnp.where(inh, d - lo, garb)
            pltpu.sync_copy(ones_v, acc16.at[idxb.at[k]], add=True)
        pltpu.sync_copy(idxb, idxl_hbm.at[c, pl.ds(cbase, BC)])

    plsc.subcore_barrier()

    rbase = s * ROWS_T
    gbase = c * HALF + s * ROWS_T
    pltpu.sync_copy(acc16.at[pl.ds(rbase, ROWS_T)],
                    deg16_hbm.at[pl.ds(gbase, ROWS_T)])


def _layer_body(src_hbm, idxl_hbm, z_hbm, dinvrow_hbm, tot_hbm,
                z_out, tot_out,
                acc, zb, srcb, idxb, rows,
                sem_g, sem_s0, sem_s1, sem_s2, sem_i):
    c = lax.axis_index("c")
    s = lax.axis_index("s")

    @pl.loop(0, ZB)
    def _fill(i):
        for j in range(D // L):
            zb[i, pl.ds(j * L, L)] = jnp.zeros((L,), jnp.float32)

    z0r = s * ZROWS_T

    @pl.loop(0, ZROWS_T // ZB)
    def _zero(i):
        pltpu.sync_copy(zb, acc.at[pl.ds(z0r + i * ZB, ZB)])

    _ztail = ZROWS_T - (ZROWS_T // ZB) * ZB
    pltpu.sync_copy(zb.at[pl.ds(0, _ztail)],
                    acc.at[pl.ds(z0r + (ZROWS_T // ZB) * ZB, _ztail)])
    plsc.subcore_barrier()

    cb0 = s * NCH_E
    pltpu.async_copy(src_hbm.at[pl.ds(cb0, BC)], srcb.at[0], sem_i)
    pltpu.async_copy(idxl_hbm.at[c, pl.ds(cb0, BC)], idxb.at[0], sem_i)

    sem_s = (sem_s0, sem_s1, sem_s2)

    def _swait(b, ib, k):
        pltpu.make_async_copy(
            rows.at[b], acc.at[idxb.at[ib, k]], sem_s[b]).wait()

    @pl.loop(0, NB)
    def _sweep(t):
        ib = t % 2
        nib = (t + 1) % 2
        cbase = s * NCH_E + t * BC
        pltpu.make_async_copy(
            src_hbm.at[pl.ds(cbase, BC)], srcb.at[ib], sem_i).wait()
        pltpu.make_async_copy(
            idxl_hbm.at[c, pl.ds(cbase, BC)], idxb.at[ib], sem_i).wait()

        @pl.when(t + 1 < NB)
        def _prefetch():
            nbase = s * NCH_E + (t + 1) * BC
            pltpu.async_copy(src_hbm.at[pl.ds(nbase, BC)], srcb.at[nib], sem_i)
            pltpu.async_copy(
                idxl_hbm.at[c, pl.ds(nbase, BC)], idxb.at[nib], sem_i)

        @pl.when(t >= 1)
        def _wprev():
            _swait(0, ib, 6)
            _swait(1, ib, 7)
        cps = [None] * BC
        cps[0] = pltpu.async_copy(z_hbm.at[srcb.at[ib, 0]], rows.at[0], sem_g)
        cps[1] = pltpu.async_copy(z_hbm.at[srcb.at[ib, 1]], rows.at[1], sem_g)
        for k in range(BC):
            b = k % KQ
            if k + 2 < BC:
                bb = (k + 2) % KQ
                if k == 0:
                    @pl.when(t >= 1)
                    def _w2():
                        _swait(bb, ib, 5)
                else:
                    _swait(bb, ib, k - 1)
                cps[k + 2] = pltpu.async_copy(
                    z_hbm.at[srcb.at[ib, k + 2]], rows.at[bb], sem_g)
            cps[k].wait()
            pltpu.async_copy(rows.at[b], acc.at[idxb.at[ib, k]],
                             sem_s[b], add=True)

    for k in (5, 6, 7):
        _swait(k % KQ, (NB - 1) % 2, k)

    plsc.subcore_barrier()

    rbase = s * ROWS_T
    gbase = c * HALF + s * ROWS_T
    r0 = rows.at[0]
    r1 = rows.at[1]

    def _scale_chunk(lbase, g0, n):
        pltpu.sync_copy(acc.at[pl.ds(lbase, n)], r0.at[pl.ds(0, n)])
        pltpu.sync_copy(dinvrow_hbm.at[pl.ds(g0, n)], r1.at[pl.ds(0, n)])

        @pl.loop(0, n)
        def _s1(r):
            for j in range(D // L):
                dvv = r1[r, pl.ds(j * L, L)]
                emb = r0[r, pl.ds(j * L, L)] * dvv
                r0[r, pl.ds(j * L, L)] = emb
                r1[r, pl.ds(j * L, L)] = emb * dvv

        pltpu.sync_copy(r1.at[pl.ds(0, n)], z_out.at[pl.ds(g0, n)])
        pltpu.sync_copy(tot_hbm.at[pl.ds(g0, n)], r1.at[pl.ds(0, n)])

        @pl.loop(0, n)
        def _s2(r):
            for j in range(D // L):
                r1[r, pl.ds(j * L, L)] = (r1[r, pl.ds(j * L, L)]
                                          + r0[r, pl.ds(j * L, L)])

        pltpu.sync_copy(r1.at[pl.ds(0, n)], tot_out.at[pl.ds(g0, n)])

    @pl.loop(0, ROWS_T // EC)
    def _out(k):
        _scale_chunk(rbase + k * EC, gbase + k * EC, EC)

    _rtail = ROWS_T - (ROWS_T // EC) * EC
    _scale_chunk(rbase + (ROWS_T // EC) * EC,
                 gbase + (ROWS_T // EC) * EC, _rtail)


def _dinv_tc_body(deg16_ref, emb0_ref, z0_ref, dinvrow_ref):
    deg = deg16_ref[:, 0:1]
    dinv = jnp.where(deg > 0.0, lax.rsqrt(jnp.maximum(deg, 1e-12)), 0.0)
    dinvrow_ref[...] = jnp.broadcast_to(dinv, (BLK, D))
    z0_ref[...] = emb0_ref[...] * dinv


@functools.lru_cache(maxsize=None)
def _build_kernels():
    mesh = plsc.VectorSubcoreMesh(
        core_axis_name="c", subcore_axis_name="s",
        num_cores=NC, num_subcores=NS)
    deg_k = pl.kernel(
        _deg_body,
        out_type=(
            jax.ShapeDtypeStruct((NPAD, L), jnp.float32),
            jax.ShapeDtypeStruct((NC, NCH_ALL, EC), jnp.int32),
        ),
        mesh=mesh,
        compiler_params=pltpu.CompilerParams(use_tc_tiling_on_sc=False),
        scratch_types=(
            pltpu.MemorySpace.VMEM_SHARED((ACC_ROWS, L), jnp.float32),
            pltpu.VMEM((EC, L), jnp.float32),
            pltpu.VMEM((ZB, L), jnp.float32),
            pltpu.VMEM((BC, EC), jnp.int32),
            pltpu.VMEM((BC, EC), jnp.int32),
        ),
    )
    layer_k = pl.kernel(
        _layer_body,
        out_type=(
            jax.ShapeDtypeStruct((NPAD, D), jnp.float32),
            jax.ShapeDtypeStruct((NPAD, D), jnp.float32),
        ),
        mesh=mesh,
        compiler_params=pltpu.CompilerParams(use_tc_tiling_on_sc=False),
        scratch_types=(
            pltpu.MemorySpace.VMEM_SHARED((ACC_ROWS, D), jnp.float32),
            pltpu.VMEM((ZB, D), jnp.float32),
            pltpu.VMEM((2, BC, EC), jnp.int32),
            pltpu.VMEM((2, BC, EC), jnp.int32),
            pltpu.VMEM((KQ, EC, D), jnp.float32),
            pltpu.SemaphoreType.DMA,
            pltpu.SemaphoreType.DMA,
            pltpu.SemaphoreType.DMA,
            pltpu.SemaphoreType.DMA,
            pltpu.SemaphoreType.DMA,
        ),
    )
    grid = (NPAD // BLK,)
    blk2 = pl.BlockSpec((BLK, D), lambda i: (i, 0))
    blk16 = pl.BlockSpec((BLK, L), lambda i: (i, 0))
    dinv_k = pl.pallas_call(
        _dinv_tc_body,
        grid=grid,
        in_specs=[blk16, blk2],
        out_specs=[blk2, blk2],
        out_shape=(
            jax.ShapeDtypeStruct((NPAD, D), jnp.float32),
            jax.ShapeDtypeStruct((NPAD, D), jnp.float32),
        ),
    )
    return deg_k, layer_k, dinv_k


def kernel(edge_index, user_weight, item_weight):
    src = edge_index[0].astype(jnp.int32)
    dst = edge_index[1].astype(jnp.int32)
    pe = EPAD - E
    pad_src = (jnp.arange(pe, dtype=jnp.int32) * 997) % N
    pad_dst = jnp.full((pe,), NPAD, dtype=jnp.int32)
    srcp = jnp.concatenate([src, pad_src]).reshape(NCH_ALL, EC)
    dstp = jnp.concatenate([dst, pad_dst]).reshape(NCH_ALL, EC)

    emb0 = jnp.concatenate([user_weight, item_weight], axis=0)
    emb0p = jnp.pad(emb0, ((0, NPAD - N), (0, 0)))

    deg_k, layer_k, dinv_k = _build_kernels()
    deg16, idxl = deg_k(dstp)
    z, dinvrow = dinv_k(deg16, emb0p)
    tot = emb0p
    for _ in range(LAYERS):
        z, tot = layer_k(srcp, idxl, z, dinvrow, tot)

    out = tot * (1.0 / ((LAYERS + 1) * (LAYERS + 1)))
    return out[:N_USERS], out[N_USERS:N]


# speedup vs baseline: 1.5596x; 1.0168x over previous
import functools

import jax
import jax.numpy as jnp
from jax import lax
from jax.experimental import pallas as pl
from jax.experimental.pallas import tpu as pltpu
from jax.experimental.pallas import tpu_sc as plsc

N_USERS = 25000
N_ITEMS = 25000
N = N_USERS + N_ITEMS
D = 64
E = 800000
LAYERS = 4

NC, NS, L = 2, 16, 16
HALF = 25088
NPAD = 2 * HALF
ROWS_T = HALF // NS
GARB = 16
ACC_ROWS = HALF + GARB
ZROWS_T = ACC_ROWS // NS
EC = 128
ES = 50176
EPAD = ES * NS
NCH_E = ES // EC
BC = 8
NB = NCH_E // BC
NCH_ALL = EPAD // EC
KQ = 3
ZB = 16
RC = 28
BLK = 512


def _deg_body(dst_hbm, deg16_hbm, idxl_hbm, acc16, ones_v, zb16, dstb, idxb,
              sem_d):
    c = lax.axis_index("c")
    s = lax.axis_index("s")

    @pl.loop(0, EC)
    def _fill(i):
        ones_v[i, :] = jnp.ones((L,), jnp.float32)

    @pl.loop(0, ZB)
    def _fillz(i):
        zb16[i, :] = jnp.zeros((L,), jnp.float32)

    z0r = s * ZROWS_T

    @pl.loop(0, ZROWS_T // ZB)
    def _zero(i):
        pltpu.sync_copy(zb16, acc16.at[pl.ds(z0r + i * ZB, ZB)])

    _ztail = ZROWS_T - (ZROWS_T // ZB) * ZB
    pltpu.sync_copy(zb16.at[pl.ds(0, _ztail)],
                    acc16.at[pl.ds(z0r + (ZROWS_T // ZB) * ZB, _ztail)])
    plsc.subcore_barrier()

    lo = c * HALF
    garb = HALF + s

    @pl.loop(0, NB)
    def _sweep(t):
        cbase = s * NCH_E + t * BC

        @pl.when(t >= 1)
        def _wprev():
            for k in range(BC):
                pltpu.make_async_copy(
                    ones_v, acc16.at[idxb.at[k]], sem_d).wait()

        pltpu.sync_copy(dst_hbm.at[pl.ds(cbase, BC)], dstb)
        for k in range(BC):
            for j in range(EC // L):
                d = dstb[k, pl.ds(j * L, L)]
                inh = (d >= lo) & (d < lo + HALF)
                idxb[k, pl.ds(j * L, L)] = jnp.where(inh, d - lo, garb)
            pltpu.async_copy(ones_v, acc16.at[idxb.at[k]], sem_d, add=True)
        pltpu.sync_copy(idxb, idxl_hbm.at[c, pl.ds(cbase, BC)])

    for k in range(BC):
        pltpu.make_async_copy(ones_v, acc16.at[idxb.at[k]], sem_d).wait()

    plsc.subcore_barrier()

    rbase = s * ROWS_T
    gbase = c * HALF + s * ROWS_T
    pltpu.sync_copy(acc16.at[pl.ds(rbase, ROWS_T)],
                    deg16_hbm.at[pl.ds(gbase, ROWS_T)])


def _layer_body(src_hbm, idxl_hbm, z_hbm, dinvrow_hbm, tot_hbm,
                z_out, tot_out,
                acc, zb, srcb, idxb, rows,
                sem_g, sem_s0, sem_s1, sem_s2, sem_i):
    c = lax.axis_index("c")
    s = lax.axis_index("s")

    @pl.loop(0, ZB)
    def _fill(i):
        for j in range(D // L):
            zb[i, pl.ds(j * L, L)] = jnp.zeros((L,), jnp.float32)

    z0r = s * ZROWS_T

    @pl.loop(0, ZROWS_T // ZB)
    def _zero(i):
        pltpu.sync_copy(zb, acc.at[pl.ds(z0r + i * ZB, ZB)])

    _ztail = ZROWS_T - (ZROWS_T // ZB) * ZB
    pltpu.sync_copy(zb.at[pl.ds(0, _ztail)],
                    acc.at[pl.ds(z0r + (ZROWS_T // ZB) * ZB, _ztail)])
    plsc.subcore_barrier()

    cb0 = s * NCH_E
    pltpu.async_copy(src_hbm.at[pl.ds(cb0, BC)], srcb.at[0], sem_i)
    pltpu.async_copy(idxl_hbm.at[c, pl.ds(cb0, BC)], idxb.at[0], sem_i)

    sem_s = (sem_s0, sem_s1, sem_s2)

    def _swait(b, ib, k):
        pltpu.make_async_copy(
            rows.at[b], acc.at[idxb.at[ib, k]], sem_s[b]).wait()

    @pl.loop(0, NB)
    def _sweep(t):
        ib = t % 2
        nib = (t + 1) % 2
        cbase = s * NCH_E + t * BC
        pltpu.make_async_copy(
            src_hbm.at[pl.ds(cbase, BC)], srcb.at[ib], sem_i).wait()
        pltpu.make_async_copy(
            idxl_hbm.at[c, pl.ds(cbase, BC)], idxb.at[ib], sem_i).wait()

        @pl.when(t + 1 < NB)
        def _prefetch():
            nbase = s * NCH_E + (t + 1) * BC
            pltpu.async_copy(src_hbm.at[pl.ds(nbase, BC)], srcb.at[nib], sem_i)
            pltpu.async_copy(
                idxl_hbm.at[c, pl.ds(nbase, BC)], idxb.at[nib], sem_i)

        @pl.when(t >= 1)
        def _wprev():
            _swait(0, ib, 6)
            _swait(1, ib, 7)
        cps = [None] * BC
        cps[0] = pltpu.async_copy(z_hbm.at[srcb.at[ib, 0]], rows.at[0], sem_g)
        cps[1] = pltpu.async_copy(z_hbm.at[srcb.at[ib, 1]], rows.at[1], sem_g)
        for k in range(BC):
            b = k % KQ
            if k + 2 < BC:
                bb = (k + 2) % KQ
                if k == 0:
                    @pl.when(t >= 1)
                    def _w2():
                        _swait(bb, ib, 5)
                else:
                    _swait(bb, ib, k - 1)
                cps[k + 2] = pltpu.async_copy(
                    z_hbm.at[srcb.at[ib, k + 2]], rows.at[bb], sem_g)
            cps[k].wait()
            pltpu.async_copy(rows.at[b], acc.at[idxb.at[ib, k]],
                             sem_s[b], add=True)

    for k in (5, 6, 7):
        _swait(k % KQ, (NB - 1) % 2, k)

    plsc.subcore_barrier()

    rbase = s * ROWS_T
    gbase = c * HALF + s * ROWS_T
    r0 = rows.at[0]
    r1 = rows.at[1]

    def _scale_chunk(lbase, g0, n):
        pltpu.sync_copy(acc.at[pl.ds(lbase, n)], r0.at[pl.ds(0, n)])
        pltpu.sync_copy(dinvrow_hbm.at[pl.ds(g0, n)], r1.at[pl.ds(0, n)])

        @pl.loop(0, n)
        def _s1(r):
            for j in range(D // L):
                dvv = r1[r, pl.ds(j * L, L)]
                emb = r0[r, pl.ds(j * L, L)] * dvv
                r0[r, pl.ds(j * L, L)] = emb
                r1[r, pl.ds(j * L, L)] = emb * dvv

        pltpu.sync_copy(r1.at[pl.ds(0, n)], z_out.at[pl.ds(g0, n)])
        pltpu.sync_copy(tot_hbm.at[pl.ds(g0, n)], r1.at[pl.ds(0, n)])

        @pl.loop(0, n)
        def _s2(r):
            for j in range(D // L):
                r1[r, pl.ds(j * L, L)] = (r1[r, pl.ds(j * L, L)]
                                          + r0[r, pl.ds(j * L, L)])

        pltpu.sync_copy(r1.at[pl.ds(0, n)], tot_out.at[pl.ds(g0, n)])

    @pl.loop(0, ROWS_T // EC)
    def _out(k):
        _scale_chunk(rbase + k * EC, gbase + k * EC, EC)

    _rtail = ROWS_T - (ROWS_T // EC) * EC
    _scale_chunk(rbase + (ROWS_T // EC) * EC,
                 gbase + (ROWS_T // EC) * EC, _rtail)


def _dinv_tc_body(deg16_ref, emb0_ref, z0_ref, dinvrow_ref):
    deg = deg16_ref[:, 0:1]
    dinv = jnp.where(deg > 0.0, lax.rsqrt(jnp.maximum(deg, 1e-12)), 0.0)
    dinvrow_ref[...] = jnp.broadcast_to(dinv, (BLK, D))
    z0_ref[...] = emb0_ref[...] * dinv


@functools.lru_cache(maxsize=None)
def _build_kernels():
    mesh = plsc.VectorSubcoreMesh(
        core_axis_name="c", subcore_axis_name="s",
        num_cores=NC, num_subcores=NS)
    deg_k = pl.kernel(
        _deg_body,
        out_type=(
            jax.ShapeDtypeStruct((NPAD, L), jnp.float32),
            jax.ShapeDtypeStruct((NC, NCH_ALL, EC), jnp.int32),
        ),
        mesh=mesh,
        compiler_params=pltpu.CompilerParams(use_tc_tiling_on_sc=False),
        scratch_types=(
            pltpu.MemorySpace.VMEM_SHARED((ACC_ROWS, L), jnp.float32),
            pltpu.VMEM((EC, L), jnp.float32),
            pltpu.VMEM((ZB, L), jnp.float32),
            pltpu.VMEM((BC, EC), jnp.int32),
            pltpu.VMEM((BC, EC), jnp.int32),
            pltpu.SemaphoreType.DMA,
        ),
    )
    layer_k = pl.kernel(
        _layer_body,
        out_type=(
            jax.ShapeDtypeStruct((NPAD, D), jnp.float32),
            jax.ShapeDtypeStruct((NPAD, D), jnp.float32),
        ),
        mesh=mesh,
        compiler_params=pltpu.CompilerParams(use_tc_tiling_on_sc=False),
        scratch_types=(
            pltpu.MemorySpace.VMEM_SHARED((ACC_ROWS, D), jnp.float32),
            pltpu.VMEM((ZB, D), jnp.float32),
            pltpu.VMEM((2, BC, EC), jnp.int32),
            pltpu.VMEM((2, BC, EC), jnp.int32),
            pltpu.VMEM((KQ, EC, D), jnp.float32),
            pltpu.SemaphoreType.DMA,
            pltpu.SemaphoreType.DMA,
            pltpu.SemaphoreType.DMA,
            pltpu.SemaphoreType.DMA,
            pltpu.SemaphoreType.DMA,
        ),
    )
    grid = (NPAD // BLK,)
    blk2 = pl.BlockSpec((BLK, D), lambda i: (i, 0))
    blk16 = pl.BlockSpec((BLK, L), lambda i: (i, 0))
    dinv_k = pl.pallas_call(
        _dinv_tc_body,
        grid=grid,
        in_specs=[blk16, blk2],
        out_specs=[blk2, blk2],
        out_shape=(
            jax.ShapeDtypeStruct((NPAD, D), jnp.float32),
            jax.ShapeDtypeStruct((NPAD, D), jnp.float32),
        ),
    )
    return deg_k, layer_k, dinv_k


def kernel(edge_index, user_weight, item_weight):
    src = edge_index[0].astype(jnp.int32)
    dst = edge_index[1].astype(jnp.int32)
    pe = EPAD - E
    pad_src = (jnp.arange(pe, dtype=jnp.int32) * 997) % N
    pad_dst = jnp.full((pe,), NPAD, dtype=jnp.int32)
    srcp = jnp.concatenate([src, pad_src]).reshape(NCH_ALL, EC)
    dstp = jnp.concatenate([dst, pad_dst]).reshape(NCH_ALL, EC)

    emb0 = jnp.concatenate([user_weight, item_weight], axis=0)
    emb0p = jnp.pad(emb0, ((0, NPAD - N), (0, 0)))

    deg_k, layer_k, dinv_k = _build_kernels()
    deg16, idxl = deg_k(dstp)
    z, dinvrow = dinv_k(deg16, emb0p)
    tot = emb0p
    for _ in range(LAYERS):
        z, tot = layer_k(srcp, idxl, z, dinvrow, tot)

    out = tot * (1.0 / ((LAYERS + 1) * (LAYERS + 1)))
    return out[:N_USERS], out[N_USERS:N]
